# baseline (speedup 1.0000x reference)
import functools
import math

import jax
import jax.numpy as jnp
from jax.experimental import pallas as pl
from jax.experimental.pallas import tpu as pltpu

G = 64
NVIS = 16
NHID = 20
NPAD = 24
DV = 32
DH = 64
DHEAD = 32
NEG = -1e30
BF = jnp.bfloat16


def _rb(x):
    return x.astype(BF).astype(jnp.float32)


def _mm(a, b):
    return jnp.matmul(a.astype(BF), b.astype(BF),
                      preferred_element_type=jnp.float32)


def _bmm_nt(a, b):
    return jax.lax.dot_general(
        a.astype(BF), b.astype(BF), (((2,), (2,)), ((0,), (0,))),
        preferred_element_type=jnp.float32)


def _bmm_nn(a, b):
    return jax.lax.dot_general(
        a.astype(BF), b.astype(BF), (((2,), (1,)), ((0,), (0,))),
        preferred_element_type=jnp.float32)


def _erf(x):
    ax = jnp.abs(x)
    t = 1.0 / (1.0 + 0.3275911 * ax)
    poly = t * (0.254829592 + t * (-0.284496736 + t * (1.421413741
                + t * (-1.453152027 + t * 1.061405429))))
    y = 1.0 - poly * jnp.exp(-ax * ax)
    return jnp.sign(x) * y


def _gelu(x):
    return 0.5 * x * (1.0 + _erf(x * (1.0 / math.sqrt(2.0))))


def _layernorm(x, g, b):
    mu = x.mean(-1, keepdims=True)
    var = ((x - mu) ** 2).mean(-1, keepdims=True)
    return (x - mu) / jnp.sqrt(var + 1e-5) * g + b


def _topk3_softmax_t(s, io):
    work = s
    m1 = None
    num = None
    den = None
    for t in range(3):
        mx = jnp.max(work, axis=1, keepdims=True)
        ism = work == mx
        first = jnp.min(jnp.where(ism, io, jnp.float32(1e9)), axis=1,
                        keepdims=True)
        pick = io == first
        if t == 0:
            m1 = mx
            num = jnp.where(pick, 1.0, 0.0)
            den = jnp.ones_like(mx)
        else:
            e = jnp.exp(mx - m1)
            num = num + jnp.where(pick, e, 0.0)
            den = den + e
        if t < 2:
            work = jnp.where(pick, NEG, work)
    return num * (1.0 / den)


def _topk3_softmax(s, io):
    work = s
    sel = None
    m1 = None
    for t in range(3):
        mx = jnp.max(work, axis=-1, keepdims=True)
        if t == 0:
            m1 = mx
        ism = work == mx
        first = jnp.min(jnp.where(ism, io, jnp.float32(1e9)), axis=-1,
                        keepdims=True)
        pick = io == first
        sel = pick if sel is None else jnp.logical_or(sel, pick)
        if t < 2:
            work = jnp.where(pick, NEG, work)
    e = jnp.where(sel, jnp.exp(s - m1), 0.0)
    return e / jnp.sum(e, axis=-1, keepdims=True)


def _body(x_ref,
          vp_w1, vp_b1, vp_w2, vp_b2, vp_emb,
          v0_qkvw, v0_qkvb, v0_ow, v0_ob, v0_g, v0_b,
          v1_qkvw, v1_qkvb, v1_ow, v1_ob, v1_g, v1_b,
          vh_w1, vh_b1, vh_w2, vh_b2, alpha,
          a_t, bh2v, ch2v,
          hp_w1, hp_b1, hp_w2, hp_b2, hp_emb,
          h0_qkvw, h0_qkvb, h0_ow, h0_ob, h0_g, h0_b,
          h1_qkvw, h1_qkvb, h1_ow, h1_ob, h1_g, h1_b,
          hh_w1, hh_b1, hh_w2, hh_b2,
          r_full,
          out_ref, va0_ref, va1_ref, ha0_ref, ha1_ref):
    x = x_ref[...]
    vis = x[:, :NVIS]

    lv = jnp.log(jnp.maximum(vis, 1e-6))
    w1 = _rb(vp_w1[...])
    h1 = (_rb(vis)[:, :, None] * w1[0][None, None, :]
          + _rb(lv)[:, :, None] * w1[1][None, None, :]
          + vp_b1[0][None, None, :])
    h1 = _gelu(h1).reshape(G * NVIS, DV)
    xv = (_mm(h1, vp_w2[...]) + vp_b2[...]).reshape(G, NVIS, DV) \
        + vp_emb[...][None]
    xv = xv.reshape(G * NVIS, DV)

    inv_s = 1.0 / math.sqrt(DHEAD)
    io16 = jax.lax.broadcasted_iota(jnp.int32, (1, NVIS, 1), 1).astype(jnp.float32)
    for (qkvw, qkvb, ow, ob, g_, b_, va_ref) in (
            (v0_qkvw, v0_qkvb, v0_ow, v0_ob, v0_g, v0_b, va0_ref),
            (v1_qkvw, v1_qkvb, v1_ow, v1_ob, v1_g, v1_b, va1_ref)):
        qkv = _mm(xv, qkvw[...]) + qkvb[...]
        q = qkv[:, :DV].reshape(G, NVIS, DV)
        k = qkv[:, DV:2 * DV].reshape(G, NVIS, DV)
        v = qkv[:, 2 * DV:].reshape(G, NVIS, DV)
        s = _bmm_nt(q, k) * inv_s
        attn = jnp.transpose(
            _topk3_softmax_t(jnp.transpose(s, (1, 2, 0)), io16), (2, 0, 1))
        va_ref[:, 0] = attn
        av = _bmm_nn(attn, v).reshape(G * NVIS, DV)
        xv = _layernorm(xv + _mm(av, ow[...]) + ob[...], g_[...], b_[...])

    hh = _gelu(_mm(xv, vh_w1[...]) + vh_b1[...])
    corr = (jnp.sum(_rb(hh).reshape(G, NVIS, NVIS) * _rb(vh_w2[...])[None],
                    axis=-1) + vh_b2[0, 0]) * alpha[0, 0]
    vis_lin = _mm(vis, a_t[...])
    ht = x[:, NVIS:NVIS + 1]
    coup = ht * bh2v[...] + (ht * ht) * ch2v[...]
    vis_lr = vis_lin + corr + coup

    la = jnp.log(jnp.maximum(x, 1e-6))
    hw1 = _rb(hp_w1[...])
    h1 = (_rb(x)[:, :, None] * hw1[0][None, None, :]
          + _rb(la)[:, :, None] * hw1[1][None, None, :]
          + hp_b1[0][None, None, :])
    h1 = _gelu(h1).reshape(G * NPAD, DH)
    xh = (_mm(h1, hp_w2[...]) + hp_b2[...]).reshape(G, NPAD, DH) \
        + hp_emb[...][None]
    xh = xh.reshape(G * NPAD, DH)

    io24 = jax.lax.broadcasted_iota(jnp.int32, (1, NPAD, 1), 1).astype(jnp.float32)
    pad_bias = jnp.where(
        jax.lax.broadcasted_iota(jnp.int32, (1, 1, NPAD), 2) >= NHID,
        NEG, 0.0)
    for (qkvw, qkvb, ow, ob, g_, b_, ha_ref) in (
            (h0_qkvw, h0_qkvb, h0_ow, h0_ob, h0_g, h0_b, ha0_ref),
            (h1_qkvw, h1_qkvb, h1_ow, h1_ob, h1_g, h1_b, ha1_ref)):
        qkv = _mm(xh, qkvw[...]) + qkvb[...]
        avs = []
        for h in range(2):
            qh = qkv[:, h * DHEAD:(h + 1) * DHEAD].reshape(G, NPAD, DHEAD)
            kh = qkv[:, DH + h * DHEAD:DH + (h + 1) * DHEAD].reshape(
                G, NPAD, DHEAD)
            vh = qkv[:, 2 * DH + h * DHEAD:2 * DH + (h + 1) * DHEAD].reshape(
                G, NPAD, DHEAD)
            s = _bmm_nt(qh, kh) * inv_s + pad_bias
            at20 = _topk3_softmax_t(
                jnp.transpose(s, (1, 2, 0))[:NHID], io24)
            attn = jnp.transpose(
                jnp.concatenate(
                    [at20, jnp.zeros((NPAD - NHID, NPAD, G), jnp.float32)],
                    axis=0),
                (2, 0, 1))
            ha_ref[:, h] = attn[:, :NHID, :NHID]
            avs.append(_bmm_nn(attn, vh))
        av = jnp.concatenate(avs, axis=-1).reshape(G * NPAD, DH)
        xh = _layernorm(xh + _mm(av, ow[...]) + ob[...], g_[...], b_[...])

    hh = _gelu(_mm(xh, hh_w1[...]) + hh_b1[...])
    val = (jnp.sum(_rb(hh).reshape(G, NPAD, DHEAD) * _rb(hh_w2[...])[None],
                   axis=-1) + hh_b2[0, 0])
    hid_lr = val[:, NVIS:NHID]

    out_ref[...] = jnp.concatenate([vis_lr, hid_lr], axis=1) + r_full[...]


def _pack(p):
    packed = [
        p['vproj_w1'].T,
        p['vproj_b1'][None, :],
        p['vproj_w2'].T,
        p['vproj_b2'][None, :],
        p['visible_node_emb'],
    ]
    for lp, nm in zip(p['visible_gat'], p['visible_norm']):
        packed += [
            jnp.concatenate([lp['q_w'].T, lp['k_w'].T, lp['v_w'].T], axis=1),
            jnp.concatenate([lp['q_b'], lp['k_b'], lp['v_b']])[None, :],
            lp['out_w'].T, lp['out_b'][None, :],
            nm['g'][None, :], nm['b'][None, :],
        ]
    packed += [
        p['vhead_w1'].T,
        p['vhead_b1'][None, :],
        p['vhead_w2'],
        p['vhead_b2'][None, :],
        jax.nn.sigmoid(p['visible_gat_alpha_raw']).reshape(1, 1),
        p['A_visible_sparse'].T,
        p['b_h2v'][None, :], p['c_h2v'][None, :],
        p['hproj_w1'].T,
        p['hproj_b1'][None, :],
        p['hproj_w2'].T,
        p['hproj_b2'][None, :],
        jnp.concatenate(
            [p['hidden_node_emb'],
             jnp.zeros((NPAD - NHID, DH), jnp.float32)], axis=0),
    ]
    for lp, nm in zip(p['hidden_gat'], p['hidden_norm']):
        packed += [
            jnp.concatenate([lp['q_w'].T, lp['k_w'].T, lp['v_w'].T], axis=1),
            jnp.concatenate([lp['q_b'], lp['k_b'], lp['v_b']])[None, :],
            lp['out_w'].T, lp['out_b'][None, :],
            nm['g'][None, :], nm['b'][None, :],
        ]
    packed += [
        p['hhead_w1'].T,
        p['hhead_b1'][None, :],
        p['hhead_w2'],
        p['hhead_b2'][None, :],
        jnp.concatenate([p['r_visible_sparse'], p['r_hidden']])[None, :],
    ]
    bf_idx = {0, 2, 5, 7, 11, 13, 17, 19, 22, 25, 27, 30, 32, 36, 38, 42, 44}
    return [a.astype(BF) if i in bf_idx else a for i, a in enumerate(packed)]


@functools.partial(jax.jit, static_argnames=('interpret',))
def _run(state, params, interpret=False):
    bb, tt, _ = state.shape
    m = bb * tt
    xs = state.reshape(m, NHID)
    xs = jnp.concatenate([xs, jnp.ones((m, NPAD - NHID), xs.dtype)], axis=1)
    packed = _pack(params)

    grid = (m // G,)
    in_specs = [pl.BlockSpec((G, NPAD), lambda i: (i, 0))]
    in_specs += [pl.BlockSpec(a.shape, lambda i: (0,) * a.ndim)
                 for a in packed]
    out_shapes = [
        jax.ShapeDtypeStruct((m, NHID), jnp.float32),
        jax.ShapeDtypeStruct((m, 1, NVIS, NVIS), jnp.float32),
        jax.ShapeDtypeStruct((m, 1, NVIS, NVIS), jnp.float32),
        jax.ShapeDtypeStruct((m, 2, NHID, NHID), jnp.float32),
        jax.ShapeDtypeStruct((m, 2, NHID, NHID), jnp.float32),
    ]
    out_specs = [
        pl.BlockSpec((G, NHID), lambda i: (i, 0)),
        pl.BlockSpec((G, 1, NVIS, NVIS), lambda i: (i, 0, 0, 0)),
        pl.BlockSpec((G, 1, NVIS, NVIS), lambda i: (i, 0, 0, 0)),
        pl.BlockSpec((G, 2, NHID, NHID), lambda i: (i, 0, 0, 0)),
        pl.BlockSpec((G, 2, NHID, NHID), lambda i: (i, 0, 0, 0)),
    ]
    out, va0, va1, ha0, ha1 = pl.pallas_call(
        _body,
        grid=grid,
        in_specs=in_specs,
        out_specs=out_specs,
        out_shape=out_shapes,
        compiler_params=pltpu.CompilerParams(
            dimension_semantics=("arbitrary",)),
        interpret=interpret,
    )(xs, *packed)
    return out.reshape(bb, tt, NHID), [va0, va1, ha0, ha1]


def kernel(state, params):
    return _run(state, params)


# speedup vs baseline: 1.1472x; 1.1472x over previous
import functools
import math

import jax
import jax.numpy as jnp
from jax.experimental import pallas as pl
from jax.experimental.pallas import tpu as pltpu

G = 128
NVIS = 16
NHID = 20
NPAD = 24
DV = 32
DH = 64
DHEAD = 32
NEG = -1e30
BF = jnp.bfloat16


def _rb(x):
    return x.astype(BF).astype(jnp.float32)


def _mm(a, b):
    return jnp.matmul(a, b.astype(BF), preferred_element_type=jnp.float32)


def _bmm_nt(a, b):
    return jax.lax.dot_general(
        a, b, (((2,), (2,)), ((0,), (0,))),
        preferred_element_type=jnp.float32)


def _bmm_nn(a, b):
    return jax.lax.dot_general(
        a, b, (((2,), (1,)), ((0,), (0,))),
        preferred_element_type=jnp.float32)


def _erf(x):
    ax = jnp.abs(x)
    t = 1.0 / (1.0 + 0.3275911 * ax)
    poly = t * (0.254829592 + t * (-0.284496736 + t * (1.421413741
                + t * (-1.453152027 + t * 1.061405429))))
    y = 1.0 - poly * jnp.exp(-ax * ax)
    return jnp.sign(x) * y


def _gelu(x):
    return 0.5 * x * (1.0 + _erf(x * (1.0 / math.sqrt(2.0))))


def _layernorm(x, g, b):
    mu = x.mean(-1, keepdims=True)
    var = ((x - mu) ** 2).mean(-1, keepdims=True)
    return (x - mu) / jnp.sqrt(var + 1e-5) * g + b


def _topk3_softmax_t(s, io):
    work = s
    m1 = None
    num = None
    den = None
    for t in range(3):
        mx = jnp.max(work, axis=1, keepdims=True)
        ism = work == mx
        first = jnp.min(jnp.where(ism, io, jnp.float32(1e9)), axis=1,
                        keepdims=True)
        pick = io == first
        if t == 0:
            m1 = mx
            num = jnp.where(pick, 1.0, 0.0)
            den = jnp.ones_like(mx)
        else:
            e = jnp.exp(mx - m1)
            num = num + jnp.where(pick, e, 0.0)
            den = den + e
        if t < 2:
            work = jnp.where(pick, NEG, work)
    return num * (1.0 / den)


def _topk3_softmax(s, io):
    work = s
    sel = None
    m1 = None
    for t in range(3):
        mx = jnp.max(work, axis=-1, keepdims=True)
        if t == 0:
            m1 = mx
        ism = work == mx
        first = jnp.min(jnp.where(ism, io, jnp.float32(1e9)), axis=-1,
                        keepdims=True)
        pick = io == first
        sel = pick if sel is None else jnp.logical_or(sel, pick)
        if t < 2:
            work = jnp.where(pick, NEG, work)
    e = jnp.where(sel, jnp.exp(s - m1), 0.0)
    return e / jnp.sum(e, axis=-1, keepdims=True)


def _body(x_ref,
          vp_w1, vp_b1, vp_w2, vp_b2, vp_emb,
          v0_qkvw, v0_qkvb, v0_ow, v0_ob, v0_g, v0_b,
          v1_qkvw, v1_qkvb, v1_ow, v1_ob, v1_g, v1_b,
          vh_w1, vh_b1, vh_w2, vh_b2, alpha,
          a_t, bh2v, ch2v,
          hp_w1, hp_b1, hp_w2, hp_b2, hp_emb,
          h0_qkvw, h0_qkvb, h0_ow, h0_ob, h0_g, h0_b,
          h1_qkvw, h1_qkvb, h1_ow, h1_ob, h1_g, h1_b,
          hh_w1, hh_b1, hh_w2, hh_b2,
          r_full,
          out_ref, va0_ref, va1_ref, ha0_ref, ha1_ref):
    x = x_ref[...]
    vis = x[:, :NVIS]

    lv = jnp.log(jnp.maximum(vis, 1e-6))
    w1 = _rb(vp_w1[...])
    h1 = (_rb(vis)[:, :, None] * w1[0][None, None, :]
          + _rb(lv)[:, :, None] * w1[1][None, None, :]
          + vp_b1[0][None, None, :])
    h1 = _gelu(h1).reshape(G * NVIS, DV)
    xv = (_mm(h1, vp_w2[...]) + vp_b2[...]).reshape(G, NVIS, DV) \
        + vp_emb[...][None]
    xv = xv.reshape(G * NVIS, DV)

    inv_s = 1.0 / math.sqrt(DHEAD)
    io16 = jax.lax.broadcasted_iota(jnp.int32, (1, NVIS, 1), 1).astype(jnp.float32)
    for (qkvw, qkvb, ow, ob, g_, b_, va_ref) in (
            (v0_qkvw, v0_qkvb, v0_ow, v0_ob, v0_g, v0_b, va0_ref),
            (v1_qkvw, v1_qkvb, v1_ow, v1_ob, v1_g, v1_b, va1_ref)):
        qkv = _mm(xv, qkvw[...]) + qkvb[...]
        q = qkv[:, :DV].reshape(G, NVIS, DV)
        k = qkv[:, DV:2 * DV].reshape(G, NVIS, DV)
        v = qkv[:, 2 * DV:].reshape(G, NVIS, DV)
        s = _bmm_nt(q, k) * inv_s
        attn = jnp.transpose(
            _topk3_softmax_t(jnp.transpose(s, (1, 2, 0)), io16), (2, 0, 1))
        va_ref[:, 0] = attn
        av = _bmm_nn(attn, v).reshape(G * NVIS, DV)
        xv = _layernorm(xv + _mm(av, ow[...]) + ob[...], g_[...], b_[...])

    hh = _gelu(_mm(xv, vh_w1[...]) + vh_b1[...])
    corr = (jnp.sum(_rb(hh).reshape(G, NVIS, NVIS) * _rb(vh_w2[...])[None],
                    axis=-1) + vh_b2[0, 0]) * alpha[0, 0]
    vis_lin = _mm(vis, a_t[...])
    ht = x[:, NVIS:NVIS + 1]
    coup = ht * bh2v[...] + (ht * ht) * ch2v[...]
    vis_lr = vis_lin + corr + coup

    la = jnp.log(jnp.maximum(x, 1e-6))
    hw1 = _rb(hp_w1[...])
    h1 = (_rb(x)[:, :, None] * hw1[0][None, None, :]
          + _rb(la)[:, :, None] * hw1[1][None, None, :]
          + hp_b1[0][None, None, :])
    h1 = _gelu(h1).reshape(G * NPAD, DH)
    xh = (_mm(h1, hp_w2[...]) + hp_b2[...]).reshape(G, NPAD, DH) \
        + hp_emb[...][None]
    xh = xh.reshape(G * NPAD, DH)

    io24 = jax.lax.broadcasted_iota(jnp.int32, (1, NPAD, 1), 1).astype(jnp.float32)
    pad_bias = jnp.where(
        jax.lax.broadcasted_iota(jnp.int32, (1, 1, NPAD), 2) >= NHID,
        NEG, 0.0)
    for (qkvw, qkvb, ow, ob, g_, b_, ha_ref) in (
            (h0_qkvw, h0_qkvb, h0_ow, h0_ob, h0_g, h0_b, ha0_ref),
            (h1_qkvw, h1_qkvb, h1_ow, h1_ob, h1_g, h1_b, ha1_ref)):
        qkv = _mm(xh, qkvw[...]) + qkvb[...]
        avs = []
        for h in range(2):
            qh = qkv[:, h * DHEAD:(h + 1) * DHEAD].reshape(G, NPAD, DHEAD)
            kh = qkv[:, DH + h * DHEAD:DH + (h + 1) * DHEAD].reshape(
                G, NPAD, DHEAD)
            vh = qkv[:, 2 * DH + h * DHEAD:2 * DH + (h + 1) * DHEAD].reshape(
                G, NPAD, DHEAD)
            s = _bmm_nt(qh, kh) * inv_s + pad_bias
            at20 = _topk3_softmax_t(
                jnp.transpose(s, (1, 2, 0))[:NHID], io24)
            attn = jnp.transpose(
                jnp.concatenate(
                    [at20, jnp.zeros((NPAD - NHID, NPAD, G), jnp.float32)],
                    axis=0),
                (2, 0, 1))
            ha_ref[:, h] = attn[:, :NHID, :NHID]
            avs.append(_bmm_nn(attn, vh))
        av = jnp.concatenate(avs, axis=-1).reshape(G * NPAD, DH)
        xh = _layernorm(xh + _mm(av, ow[...]) + ob[...], g_[...], b_[...])

    hh = _gelu(_mm(xh, hh_w1[...]) + hh_b1[...])
    val = (jnp.sum(_rb(hh).reshape(G, NPAD, DHEAD) * _rb(hh_w2[...])[None],
                   axis=-1) + hh_b2[0, 0])
    hid_lr = val[:, NVIS:NHID]

    out_ref[...] = jnp.concatenate([vis_lr, hid_lr], axis=1) + r_full[...]


def _pack(p):
    packed = [
        p['vproj_w1'].T,
        p['vproj_b1'][None, :],
        p['vproj_w2'].T,
        p['vproj_b2'][None, :],
        p['visible_node_emb'],
    ]
    for lp, nm in zip(p['visible_gat'], p['visible_norm']):
        packed += [
            jnp.concatenate([lp['q_w'].T, lp['k_w'].T, lp['v_w'].T], axis=1),
            jnp.concatenate([lp['q_b'], lp['k_b'], lp['v_b']])[None, :],
            lp['out_w'].T, lp['out_b'][None, :],
            nm['g'][None, :], nm['b'][None, :],
        ]
    packed += [
        p['vhead_w1'].T,
        p['vhead_b1'][None, :],
        p['vhead_w2'],
        p['vhead_b2'][None, :],
        jax.nn.sigmoid(p['visible_gat_alpha_raw']).reshape(1, 1),
        p['A_visible_sparse'].T,
        p['b_h2v'][None, :], p['c_h2v'][None, :],
        p['hproj_w1'].T,
        p['hproj_b1'][None, :],
        p['hproj_w2'].T,
        p['hproj_b2'][None, :],
        jnp.concatenate(
            [p['hidden_node_emb'],
             jnp.zeros((NPAD - NHID, DH), jnp.float32)], axis=0),
    ]
    for lp, nm in zip(p['hidden_gat'], p['hidden_norm']):
        packed += [
            jnp.concatenate([lp['q_w'].T, lp['k_w'].T, lp['v_w'].T], axis=1),
            jnp.concatenate([lp['q_b'], lp['k_b'], lp['v_b']])[None, :],
            lp['out_w'].T, lp['out_b'][None, :],
            nm['g'][None, :], nm['b'][None, :],
        ]
    packed += [
        p['hhead_w1'].T,
        p['hhead_b1'][None, :],
        p['hhead_w2'],
        p['hhead_b2'][None, :],
        jnp.concatenate([p['r_visible_sparse'], p['r_hidden']])[None, :],
    ]
    bf_idx = {0, 2, 5, 7, 11, 13, 17, 19, 22, 25, 27, 30, 32, 36, 38, 42, 44}
    return [a.astype(BF) if i in bf_idx else a for i, a in enumerate(packed)]


@functools.partial(jax.jit, static_argnames=('interpret',))
def _run(state, params, interpret=False):
    bb, tt, _ = state.shape
    m = bb * tt
    xs = state.reshape(m, NHID)
    xs = jnp.concatenate([xs, jnp.ones((m, NPAD - NHID), xs.dtype)], axis=1)
    packed = _pack(params)

    grid = (m // G,)
    in_specs = [pl.BlockSpec((G, NPAD), lambda i: (i, 0))]
    in_specs += [pl.BlockSpec(a.shape, lambda i: (0,) * a.ndim)
                 for a in packed]
    out_shapes = [
        jax.ShapeDtypeStruct((m, NHID), jnp.float32),
        jax.ShapeDtypeStruct((m, 1, NVIS, NVIS), jnp.float32),
        jax.ShapeDtypeStruct((m, 1, NVIS, NVIS), jnp.float32),
        jax.ShapeDtypeStruct((m, 2, NHID, NHID), jnp.float32),
        jax.ShapeDtypeStruct((m, 2, NHID, NHID), jnp.float32),
    ]
    out_specs = [
        pl.BlockSpec((G, NHID), lambda i: (i, 0)),
        pl.BlockSpec((G, 1, NVIS, NVIS), lambda i: (i, 0, 0, 0)),
        pl.BlockSpec((G, 1, NVIS, NVIS), lambda i: (i, 0, 0, 0)),
        pl.BlockSpec((G, 2, NHID, NHID), lambda i: (i, 0, 0, 0)),
        pl.BlockSpec((G, 2, NHID, NHID), lambda i: (i, 0, 0, 0)),
    ]
    out, va0, va1, ha0, ha1 = pl.pallas_call(
        _body,
        grid=grid,
        in_specs=in_specs,
        out_specs=out_specs,
        out_shape=out_shapes,
        compiler_params=pltpu.CompilerParams(
            dimension_semantics=("arbitrary",)),
        interpret=interpret,
    )(xs, *packed)
    return out.reshape(bb, tt, NHID), [va0, va1, ha0, ha1]


def kernel(state, params):
    return _run(state, params)


# speedup vs baseline: 1.1687x; 1.0188x over previous
import functools
import math

import jax
import jax.numpy as jnp
from jax.experimental import pallas as pl
from jax.experimental.pallas import tpu as pltpu

G = 128
NVIS = 16
NHID = 20
NPAD = 24
DV = 32
DH = 64
DHEAD = 32
NEG = -1e30
BF = jnp.bfloat16


def _rb(x):
    return x.astype(BF).astype(jnp.float32)


def _mm(a, b):
    return jnp.matmul(a.astype(BF), b.astype(BF),
                      preferred_element_type=jnp.float32)


def _bmm_nt(a, b):
    return jax.lax.dot_general(
        a.astype(BF), b.astype(BF), (((2,), (2,)), ((0,), (0,))),
        preferred_element_type=jnp.float32)


def _bmm_nn(a, b):
    return jax.lax.dot_general(
        a.astype(BF), b.astype(BF), (((2,), (1,)), ((0,), (0,))),
        preferred_element_type=jnp.float32)


def _erf(x):
    ax = jnp.abs(x)
    t = 1.0 / (1.0 + 0.3275911 * ax)
    poly = t * (0.254829592 + t * (-0.284496736 + t * (1.421413741
                + t * (-1.453152027 + t * 1.061405429))))
    y = 1.0 - poly * jnp.exp(-ax * ax)
    return jnp.sign(x) * y


def _gelu(x):
    return 0.5 * x * (1.0 + _erf(x * (1.0 / math.sqrt(2.0))))


def _layernorm(x, g, b):
    mu = x.mean(-1, keepdims=True)
    var = ((x - mu) ** 2).mean(-1, keepdims=True)
    return (x - mu) / jnp.sqrt(var + 1e-5) * g + b


def _topk3_softmax_t(s, io):
    work = s
    m1 = None
    num = None
    den = None
    for t in range(3):
        mx = jnp.max(work, axis=1, keepdims=True)
        ism = work == mx
        first = jnp.min(jnp.where(ism, io, jnp.float32(1e9)), axis=1,
                        keepdims=True)
        pick = io == first
        if t == 0:
            m1 = mx
            num = jnp.where(pick, 1.0, 0.0)
            den = jnp.ones_like(mx)
        else:
            e = jnp.exp(mx - m1)
            num = num + jnp.where(pick, e, 0.0)
            den = den + e
        if t < 2:
            work = jnp.where(pick, NEG, work)
    return num * (1.0 / den)


def _topk3_softmax(s, io):
    work = s
    sel = None
    m1 = None
    for t in range(3):
        mx = jnp.max(work, axis=-1, keepdims=True)
        if t == 0:
            m1 = mx
        ism = work == mx
        first = jnp.min(jnp.where(ism, io, jnp.float32(1e9)), axis=-1,
                        keepdims=True)
        pick = io == first
        sel = pick if sel is None else jnp.logical_or(sel, pick)
        if t < 2:
            work = jnp.where(pick, NEG, work)
    e = jnp.where(sel, jnp.exp(s - m1), 0.0)
    return e / jnp.sum(e, axis=-1, keepdims=True)


def _body(x_ref,
          vp_w1, vp_b1, vp_w2, vp_b2, vp_emb,
          v0_qkvw, v0_qkvb, v0_ow, v0_ob, v0_g, v0_b,
          v1_qkvw, v1_qkvb, v1_ow, v1_ob, v1_g, v1_b,
          vh_w1, vh_b1, vh_w2, vh_b2, alpha,
          a_t, bh2v, ch2v,
          hp_w1, hp_b1, hp_w2, hp_b2, hp_emb,
          h0_qkvw, h0_qkvb, h0_ow, h0_ob, h0_g, h0_b,
          h1_qkvw, h1_qkvb, h1_ow, h1_ob, h1_g, h1_b,
          hh_w1, hh_b1, hh_w2, hh_b2,
          r_full,
          out_ref, va0_ref, va1_ref, ha0_ref, ha1_ref):
    x = x_ref[...]
    vis = x[:, :NVIS]

    lv = jnp.log(jnp.maximum(vis, 1e-6))
    w1 = _rb(vp_w1[...])
    h1 = (_rb(vis)[:, :, None] * w1[0][None, None, :]
          + _rb(lv)[:, :, None] * w1[1][None, None, :]
          + vp_b1[0][None, None, :])
    h1 = _gelu(h1).reshape(G * NVIS, DV)
    xv = (_mm(h1, vp_w2[...]) + vp_b2[...]).reshape(G, NVIS, DV) \
        + vp_emb[...][None]
    xv = xv.reshape(G * NVIS, DV)

    inv_s = 1.0 / math.sqrt(DHEAD)
    io16 = jax.lax.broadcasted_iota(jnp.int32, (1, NVIS, 1), 1).astype(jnp.float32)
    for (qkvw, qkvb, ow, ob, g_, b_, va_ref) in (
            (v0_qkvw, v0_qkvb, v0_ow, v0_ob, v0_g, v0_b, va0_ref),
            (v1_qkvw, v1_qkvb, v1_ow, v1_ob, v1_g, v1_b, va1_ref)):
        qkv = _mm(xv, qkvw[...]) + qkvb[...]
        q = qkv[:, :DV].reshape(G, NVIS, DV)
        k = qkv[:, DV:2 * DV].reshape(G, NVIS, DV)
        v = qkv[:, 2 * DV:].reshape(G, NVIS, DV)
        s = _bmm_nt(q, k) * inv_s
        attn = jnp.transpose(
            _topk3_softmax_t(jnp.transpose(s, (1, 2, 0)), io16), (2, 0, 1))
        va_ref[:, 0] = attn
        av = _bmm_nn(attn, v).reshape(G * NVIS, DV)
        xv = _layernorm(xv + _mm(av, ow[...]) + ob[...], g_[...], b_[...])

    hh = _gelu(_mm(xv, vh_w1[...]) + vh_b1[...])
    corr = (jnp.sum(_rb(hh).reshape(G, NVIS, NVIS) * _rb(vh_w2[...])[None],
                    axis=-1) + vh_b2[0, 0]) * alpha[0, 0]
    vis_lin = _mm(vis, a_t[...])
    ht = x[:, NVIS:NVIS + 1]
    coup = ht * bh2v[...] + (ht * ht) * ch2v[...]
    vis_lr = vis_lin + corr + coup

    la = jnp.log(jnp.maximum(x, 1e-6))
    hw1 = _rb(hp_w1[...])
    h1 = (_rb(x)[:, :, None] * hw1[0][None, None, :]
          + _rb(la)[:, :, None] * hw1[1][None, None, :]
          + hp_b1[0][None, None, :])
    h1 = _gelu(h1).reshape(G * NPAD, DH)
    xh = (_mm(h1, hp_w2[...]) + hp_b2[...]).reshape(G, NPAD, DH) \
        + hp_emb[...][None]
    xh = xh.reshape(G * NPAD, DH)

    io24 = jax.lax.broadcasted_iota(jnp.int32, (1, NPAD, 1), 1).astype(jnp.float32)
    pad_bias = jnp.where(
        jax.lax.broadcasted_iota(jnp.int32, (1, 1, NPAD), 2) >= NHID,
        NEG, 0.0)
    for (qkvw, qkvb, ow, ob, g_, b_, ha_ref) in (
            (h0_qkvw, h0_qkvb, h0_ow, h0_ob, h0_g, h0_b, ha0_ref),
            (h1_qkvw, h1_qkvb, h1_ow, h1_ob, h1_g, h1_b, ha1_ref)):
        qkv = _mm(xh, qkvw[...]) + qkvb[...]
        avs = []
        for h in range(2):
            qh = qkv[:, h * DHEAD:(h + 1) * DHEAD].reshape(G, NPAD, DHEAD)
            kh = qkv[:, DH + h * DHEAD:DH + (h + 1) * DHEAD].reshape(
                G, NPAD, DHEAD)
            vh = qkv[:, 2 * DH + h * DHEAD:2 * DH + (h + 1) * DHEAD].reshape(
                G, NPAD, DHEAD)
            s = _bmm_nt(qh, kh) * inv_s + pad_bias
            at20 = _topk3_softmax_t(
                jnp.transpose(s, (1, 2, 0))[:NHID], io24)
            attn = jnp.transpose(
                jnp.concatenate(
                    [at20, jnp.zeros((NPAD - NHID, NPAD, G), jnp.float32)],
                    axis=0),
                (2, 0, 1))
            ha_ref[:, h] = attn[:, :NHID, :NHID]
            avs.append(_bmm_nn(attn, vh))
        av = jnp.concatenate(avs, axis=-1).reshape(G * NPAD, DH)
        xh = _layernorm(xh + _mm(av, ow[...]) + ob[...], g_[...], b_[...])

    hh = _gelu(_mm(xh, hh_w1[...]) + hh_b1[...])
    val = (jnp.sum(_rb(hh).reshape(G, NPAD, DHEAD) * _rb(hh_w2[...])[None],
                   axis=-1) + hh_b2[0, 0])
    hid_lr = val[:, NVIS:NHID]

    out_ref[...] = jnp.concatenate([vis_lr, hid_lr], axis=1) + r_full[...]


def _pack(p):
    packed = [
        p['vproj_w1'].T,
        p['vproj_b1'][None, :],
        p['vproj_w2'].T,
        p['vproj_b2'][None, :],
        p['visible_node_emb'],
    ]
    for lp, nm in zip(p['visible_gat'], p['visible_norm']):
        packed += [
            jnp.concatenate([lp['q_w'].T, lp['k_w'].T, lp['v_w'].T], axis=1),
            jnp.concatenate([lp['q_b'], lp['k_b'], lp['v_b']])[None, :],
            lp['out_w'].T, lp['out_b'][None, :],
            nm['g'][None, :], nm['b'][None, :],
        ]
    packed += [
        p['vhead_w1'].T,
        p['vhead_b1'][None, :],
        p['vhead_w2'],
        p['vhead_b2'][None, :],
        jax.nn.sigmoid(p['visible_gat_alpha_raw']).reshape(1, 1),
        p['A_visible_sparse'].T,
        p['b_h2v'][None, :], p['c_h2v'][None, :],
        p['hproj_w1'].T,
        p['hproj_b1'][None, :],
        p['hproj_w2'].T,
        p['hproj_b2'][None, :],
        jnp.concatenate(
            [p['hidden_node_emb'],
             jnp.zeros((NPAD - NHID, DH), jnp.float32)], axis=0),
    ]
    for lp, nm in zip(p['hidden_gat'], p['hidden_norm']):
        packed += [
            jnp.concatenate([lp['q_w'].T, lp['k_w'].T, lp['v_w'].T], axis=1),
            jnp.concatenate([lp['q_b'], lp['k_b'], lp['v_b']])[None, :],
            lp['out_w'].T, lp['out_b'][None, :],
            nm['g'][None, :], nm['b'][None, :],
        ]
    packed += [
        p['hhead_w1'].T,
        p['hhead_b1'][None, :],
        p['hhead_w2'],
        p['hhead_b2'][None, :],
        jnp.concatenate([p['r_visible_sparse'], p['r_hidden']])[None, :],
    ]
    bf_idx = {0, 2, 5, 7, 11, 13, 17, 19, 22, 25, 27, 30, 32, 36, 38, 42, 44}
    return [a.astype(BF) if i in bf_idx else a for i, a in enumerate(packed)]


@jax.jit
def _run(state, params):
    bb, tt, _ = state.shape
    m = bb * tt
    xs = state.reshape(m, NHID)
    xs = jnp.concatenate([xs, jnp.ones((m, NPAD - NHID), xs.dtype)], axis=1)
    packed = _pack(params)

    grid = (m // G,)
    in_specs = [pl.BlockSpec((G, NPAD), lambda i: (i, 0))]
    in_specs += [pl.BlockSpec(a.shape, lambda i: (0,) * a.ndim)
                 for a in packed]
    out_shapes = [
        jax.ShapeDtypeStruct((m, NHID), jnp.float32),
        jax.ShapeDtypeStruct((m, 1, NVIS, NVIS), jnp.float32),
        jax.ShapeDtypeStruct((m, 1, NVIS, NVIS), jnp.float32),
        jax.ShapeDtypeStruct((m, 2, NHID, NHID), jnp.float32),
        jax.ShapeDtypeStruct((m, 2, NHID, NHID), jnp.float32),
    ]
    out_specs = [
        pl.BlockSpec((G, NHID), lambda i: (i, 0)),
        pl.BlockSpec((G, 1, NVIS, NVIS), lambda i: (i, 0, 0, 0)),
        pl.BlockSpec((G, 1, NVIS, NVIS), lambda i: (i, 0, 0, 0)),
        pl.BlockSpec((G, 2, NHID, NHID), lambda i: (i, 0, 0, 0)),
        pl.BlockSpec((G, 2, NHID, NHID), lambda i: (i, 0, 0, 0)),
    ]
    out, va0, va1, ha0, ha1 = pl.pallas_call(
        _body,
        grid=grid,
        in_specs=in_specs,
        out_specs=out_specs,
        out_shape=out_shapes,
        compiler_params=pltpu.CompilerParams(
            dimension_semantics=("arbitrary",)),
    )(xs, *packed)
    return out.reshape(bb, tt, NHID), [va0, va1, ha0, ha1]


def kernel(state, params):
    return _run(state, params)
